# Initial kernel scaffold; baseline (speedup 1.0000x reference)
import jax
import jax.numpy as jnp
from jax.experimental import pallas as pl


def kernel(x, edge_index, W, a_src, a_dst, bias):
    raise NotImplementedError("write your pallas kernel here")


# speedup vs baseline: 288.8728x; 288.8728x over previous
import functools

import jax
import jax.numpy as jnp
from jax import lax
from jax.experimental import pallas as pl
from jax.experimental.pallas import tpu as pltpu
from jax.experimental.pallas import tpu_sc as plsc

N_NODES = 100000
N_EDGES = 1600000
HEADS = 8
OUT_CH = 8

NC = 2
NS = 16
NW = NC * NS
LANES = 16

BLK = 512
CHUNK = 128
NCHUNK = BLK // CHUNK
GPC = CHUNK // LANES
NBLOCKS = N_EDGES // BLK
BLK_PER, BLK_REM = divmod(NBLOCKS, NW)
ROWS_PER_TILE = N_NODES // NS
ZCOPY = 512
ZTAIL = ROWS_PER_TILE - (ROWS_PER_TILE // ZCOPY) * ZCOPY


def _sc_body(x_hbm, src_hbm, dst_hbm, sv_hbm, dv_hbm, part_hbm,
             srcbuf, dstbuf, xsbuf, xdbuf, contrib, svv, dvv,
             acc, sem_g, sem_s0, sem_s1):
    c = lax.axis_index("c")
    s = lax.axis_index("s")
    wid = s * NC + c
    sem_s = [sem_s0, sem_s1]

    pltpu.sync_copy(sv_hbm, svv)
    pltpu.sync_copy(dv_hbm, dvv)

    def _zero_rows(i, _):
        contrib[0, i, :] = jnp.zeros((LANES,), jnp.float32)
        return 0
    lax.fori_loop(0, BLK, _zero_rows, 0)

    def _zero_acc(j, _):
        pltpu.sync_copy(contrib.at[0],
                        acc.at[pl.ds(s * ROWS_PER_TILE + j * ZCOPY, ZCOPY)])
        return 0
    lax.fori_loop(0, ROWS_PER_TILE // ZCOPY, _zero_acc, 0)
    pltpu.sync_copy(contrib.at[0, pl.ds(0, ZTAIL)],
                    acc.at[pl.ds(s * ROWS_PER_TILE
                                 + (ROWS_PER_TILE // ZCOPY) * ZCOPY, ZTAIL)])
    plsc.subcore_barrier()

    svh = [svv[h] for h in range(HEADS)]
    dvh = [dvv[h] for h in range(HEADS)]
    iota = lax.iota(jnp.int32, LANES)
    cols_d = [jnp.full((LANES,), h, jnp.int32) for h in range(HEADS)]
    cols_n = [jnp.full((LANES,), HEADS + h, jnp.int32) for h in range(HEADS)]

    nblk = BLK_PER + jnp.where(wid < BLK_REM, 1, 0)
    start = wid * BLK_PER + jnp.minimum(wid, BLK_REM)

    def _drain_scatter(j):
        pltpu.make_async_copy(contrib.at[j], acc.at[pl.ds(0, BLK)],
                              sem_s[j]).wait()

    def _do_block(b, j):
        @pl.when(b >= 2)
        def _():
            _drain_scatter(j)
        ebase = (start + b) * BLK
        cps = [pltpu.async_copy(src_hbm.at[pl.ds(ebase, BLK)],
                                srcbuf.at[j], sem_g)]
        for k in range(NCHUNK):
            cps.append(pltpu.async_copy(
                dst_hbm.at[pl.ds(ebase + k * CHUNK, CHUNK)],
                dstbuf.at[j, k], sem_g))
        for cp in cps:
            cp.wait()
        cps = []
        for k in range(NCHUNK):
            cps.append(pltpu.async_copy(
                x_hbm.at[srcbuf.at[j, pl.ds(k * CHUNK, CHUNK)]],
                xsbuf.at[j, k], sem_g))
            cps.append(pltpu.async_copy(
                x_hbm.at[dstbuf.at[j, k]], xdbuf.at[j, k], sem_g))
        for cp in cps:
            cp.wait()
        for g in range(BLK // LANES):
            k, i = divmod(g, GPC)
            xs = xsbuf[j, k, pl.ds(i * LANES, LANES)]
            xd = xdbuf[j, k, pl.ds(i * LANES, LANES)]
            row_idx = iota + (g * LANES)
            for h in range(HEADS):
                e = xs * svh[h] + xd * dvh[h]
                e = jnp.maximum(e, e * jnp.float32(0.2))
                ex = jnp.exp(e)
                plsc.store_scatter(contrib.at[j], [row_idx, cols_d[h]], ex)
                plsc.store_scatter(contrib.at[j], [row_idx, cols_n[h]],
                                   ex * xs)
        for k in range(NCHUNK):
            pltpu.async_copy(contrib.at[j, pl.ds(k * CHUNK, CHUNK)],
                             acc.at[dstbuf.at[j, k]], sem_s[j], add=True)

    def _pair(p, _):
        _do_block(2 * p, 0)
        _do_block(2 * p + 1, 1)
        return 0

    lax.fori_loop(0, nblk // 2, _pair, 0)

    @pl.when(nblk % 2 == 1)
    def _():
        _do_block(nblk - 1, 0)

    _drain_scatter(0)

    @pl.when(nblk >= 2)
    def _():
        _drain_scatter(1)

    plsc.subcore_barrier()

    pltpu.sync_copy(acc.at[pl.ds(s * ROWS_PER_TILE, ROWS_PER_TILE)],
                    part_hbm.at[c, pl.ds(s * ROWS_PER_TILE, ROWS_PER_TILE)])


@functools.partial(
    pl.kernel,
    mesh=plsc.VectorSubcoreMesh(core_axis_name="c", subcore_axis_name="s"),
    compiler_params=pltpu.CompilerParams(use_tc_tiling_on_sc=False,
                                         needs_layout_passes=False),
    out_type=jax.ShapeDtypeStruct((NC, N_NODES, 2 * HEADS), jnp.float32),
    scratch_types=[
        pltpu.VMEM((2, BLK), jnp.int32),
        pltpu.VMEM((2, NCHUNK, CHUNK), jnp.int32),
        pltpu.VMEM((2, NCHUNK, CHUNK), jnp.float32),
        pltpu.VMEM((2, NCHUNK, CHUNK), jnp.float32),
        pltpu.VMEM((2, BLK, 2 * HEADS), jnp.float32),
        pltpu.VMEM((HEADS, LANES), jnp.float32),
        pltpu.VMEM((HEADS, LANES), jnp.float32),
        pltpu.VMEM_SHARED((N_NODES, 2 * HEADS), jnp.float32),
        pltpu.SemaphoreType.DMA,
        pltpu.SemaphoreType.DMA,
        pltpu.SemaphoreType.DMA,
    ],
)
def _sc_edge_pass(x_hbm, src_hbm, dst_hbm, sv_hbm, dv_hbm, part_hbm,
                  srcbuf, dstbuf, xsbuf, xdbuf, contrib, svv, dvv,
                  acc, sem_g, sem_s0, sem_s1):
    _sc_body(x_hbm, src_hbm, dst_hbm, sv_hbm, dv_hbm, part_hbm,
             srcbuf, dstbuf, xsbuf, xdbuf, contrib, svv, dvv,
             acc, sem_g, sem_s0, sem_s1)


FIN_BN = 1000


def _finalize_body(p_ref, s_ref, b_ref, o_ref):
    p = p_ref[0] + p_ref[1]
    den = p[:, :HEADS]
    num = p[:, HEADS:]
    g = num / (den + jnp.float32(1e-16))
    o = jnp.dot(g, s_ref[...], preferred_element_type=jnp.float32)
    o_ref[...] = jnp.maximum(o + b_ref[...], jnp.float32(0.0))


_finalize = pl.pallas_call(
    _finalize_body,
    out_shape=jax.ShapeDtypeStruct((N_NODES, HEADS * OUT_CH), jnp.float32),
    grid=(N_NODES // FIN_BN,),
    in_specs=[
        pl.BlockSpec((NC, FIN_BN, 2 * HEADS), lambda i: (0, i, 0)),
        pl.BlockSpec((HEADS, HEADS * OUT_CH), lambda i: (0, 0)),
        pl.BlockSpec((1, HEADS * OUT_CH), lambda i: (0, 0)),
    ],
    out_specs=pl.BlockSpec((FIN_BN, HEADS * OUT_CH), lambda i: (i, 0)),
)


def kernel(x, edge_index, W, a_src, a_dst, bias):
    xf = x.reshape(N_NODES)
    src = edge_index[0]
    dst = edge_index[1]
    Wr = W.reshape(HEADS, OUT_CH)
    s = jnp.sum(Wr * a_src, axis=1)
    d = jnp.sum(Wr * a_dst, axis=1)
    sv = jnp.broadcast_to(s[:, None], (HEADS, LANES))
    dv = jnp.broadcast_to(d[:, None], (HEADS, LANES))
    part = _sc_edge_pass(xf, src, dst, sv, dv)
    S = jnp.repeat(jnp.eye(HEADS, dtype=jnp.float32), OUT_CH, axis=1) * W
    out = _finalize(part, S, bias.reshape(1, HEADS * OUT_CH))
    return out


# speedup vs baseline: 407.4061x; 1.4103x over previous
import functools

import jax
import jax.numpy as jnp
from jax import lax
from jax.experimental import pallas as pl
from jax.experimental.pallas import tpu as pltpu
from jax.experimental.pallas import tpu_sc as plsc

N_NODES = 100000
N_EDGES = 1600000
HEADS = 8
OUT_CH = 8

NC = 2
NS = 16
NW = NC * NS
LANES = 16

BLK = 1280
CHUNK = 128
NCHUNK = BLK // CHUNK
NBLOCKS = N_EDGES // BLK
BLK_PER, BLK_REM = divmod(NBLOCKS, NW)
ROWS_PER_TILE = N_NODES // NS
NZCOPY = ROWS_PER_TILE // BLK
ZTAIL = ROWS_PER_TILE - NZCOPY * BLK


def _sc_body(x_hbm, src_hbm, dst_hbm, sv_hbm, dv_hbm, part_hbm,
             srcbuf, dstbuf, xsbuf, xdbuf, contrib, svv, dvv,
             acc, sem_g, sem_s):
    c = lax.axis_index("c")
    s = lax.axis_index("s")
    wid = s * NC + c

    pltpu.sync_copy(sv_hbm, svv)
    pltpu.sync_copy(dv_hbm, dvv)

    def _zero_rows(i, _):
        contrib[i, :] = jnp.zeros((LANES,), jnp.float32)
        return 0
    lax.fori_loop(0, BLK, _zero_rows, 0)

    def _zero_acc(j, _):
        pltpu.sync_copy(contrib,
                        acc.at[pl.ds(s * ROWS_PER_TILE + j * BLK, BLK)])
        return 0
    lax.fori_loop(0, NZCOPY, _zero_acc, 0)
    pltpu.sync_copy(contrib.at[pl.ds(0, ZTAIL)],
                    acc.at[pl.ds(s * ROWS_PER_TILE + NZCOPY * BLK, ZTAIL)])
    plsc.subcore_barrier()

    svh = [svv[h] for h in range(HEADS)]
    dvh = [dvv[h] for h in range(HEADS)]
    iota = lax.iota(jnp.int32, LANES)
    cols_d = [jnp.full((LANES,), h, jnp.int32) for h in range(HEADS)]
    cols_n = [jnp.full((LANES,), HEADS + h, jnp.int32) for h in range(HEADS)]

    nblk = BLK_PER + jnp.where(wid < BLK_REM, 1, 0)
    start = wid * BLK_PER + jnp.minimum(wid, BLK_REM)

    def _block(b, _):
        ebase = (start + b) * BLK
        cps = [pltpu.async_copy(src_hbm.at[pl.ds(ebase, BLK)], srcbuf, sem_g)]
        for k in range(NCHUNK):
            cps.append(pltpu.async_copy(
                dst_hbm.at[pl.ds(ebase + k * CHUNK, CHUNK)],
                dstbuf.at[k], sem_g))
        for cp in cps:
            cp.wait()
        cps = []
        for k in range(NCHUNK):
            cps.append(pltpu.async_copy(
                x_hbm.at[srcbuf.at[pl.ds(k * CHUNK, CHUNK)]],
                xsbuf.at[pl.ds(k * CHUNK, CHUNK)], sem_g))
            cps.append(pltpu.async_copy(
                x_hbm.at[dstbuf.at[k]],
                xdbuf.at[pl.ds(k * CHUNK, CHUNK)], sem_g))
        for cp in cps:
            cp.wait()

        def _group(g, _):
            xs = xsbuf[pl.ds(g * LANES, LANES)]
            xd = xdbuf[pl.ds(g * LANES, LANES)]
            row_idx = iota + g * LANES
            for h in range(HEADS):
                e = xs * svh[h] + xd * dvh[h]
                e = jnp.maximum(e, e * jnp.float32(0.2))
                ex = jnp.exp(e)
                plsc.store_scatter(contrib, [row_idx, cols_d[h]], ex)
                plsc.store_scatter(contrib, [row_idx, cols_n[h]], ex * xs)
            return 0

        lax.fori_loop(0, BLK // LANES, _group, 0)

        for k in range(NCHUNK):
            pltpu.async_copy(contrib.at[pl.ds(k * CHUNK, CHUNK)],
                             acc.at[dstbuf.at[k]], sem_s, add=True)
        pltpu.make_async_copy(contrib, acc.at[pl.ds(0, BLK)], sem_s).wait()
        return 0

    lax.fori_loop(0, nblk, _block, 0)
    plsc.subcore_barrier()

    pltpu.sync_copy(acc.at[pl.ds(s * ROWS_PER_TILE, ROWS_PER_TILE)],
                    part_hbm.at[c, pl.ds(s * ROWS_PER_TILE, ROWS_PER_TILE)])


@functools.partial(
    pl.kernel,
    mesh=plsc.VectorSubcoreMesh(core_axis_name="c", subcore_axis_name="s"),
    compiler_params=pltpu.CompilerParams(use_tc_tiling_on_sc=False,
                                         needs_layout_passes=False),
    out_type=jax.ShapeDtypeStruct((NC, N_NODES, 2 * HEADS), jnp.float32),
    scratch_types=[
        pltpu.VMEM((BLK,), jnp.int32),
        pltpu.VMEM((NCHUNK, CHUNK), jnp.int32),
        pltpu.VMEM((BLK,), jnp.float32),
        pltpu.VMEM((BLK,), jnp.float32),
        pltpu.VMEM((BLK, 2 * HEADS), jnp.float32),
        pltpu.VMEM((HEADS, LANES), jnp.float32),
        pltpu.VMEM((HEADS, LANES), jnp.float32),
        pltpu.VMEM_SHARED((N_NODES, 2 * HEADS), jnp.float32),
        pltpu.SemaphoreType.DMA,
        pltpu.SemaphoreType.DMA,
    ],
)
def _sc_edge_pass(x_hbm, src_hbm, dst_hbm, sv_hbm, dv_hbm, part_hbm,
                  srcbuf, dstbuf, xsbuf, xdbuf, contrib, svv, dvv,
                  acc, sem_g, sem_s):
    _sc_body(x_hbm, src_hbm, dst_hbm, sv_hbm, dv_hbm, part_hbm,
             srcbuf, dstbuf, xsbuf, xdbuf, contrib, svv, dvv,
             acc, sem_g, sem_s)


FIN_BN = 1000


def _finalize_body(p_ref, s_ref, b_ref, o_ref):
    p = p_ref[0] + p_ref[1]
    den = p[:, :HEADS]
    num = p[:, HEADS:]
    g = num / (den + jnp.float32(1e-16))
    o = jnp.dot(g, s_ref[...], preferred_element_type=jnp.float32)
    o_ref[...] = jnp.maximum(o + b_ref[...], jnp.float32(0.0))


_finalize = pl.pallas_call(
    _finalize_body,
    out_shape=jax.ShapeDtypeStruct((N_NODES, HEADS * OUT_CH), jnp.float32),
    grid=(N_NODES // FIN_BN,),
    in_specs=[
        pl.BlockSpec((NC, FIN_BN, 2 * HEADS), lambda i: (0, i, 0)),
        pl.BlockSpec((HEADS, HEADS * OUT_CH), lambda i: (0, 0)),
        pl.BlockSpec((1, HEADS * OUT_CH), lambda i: (0, 0)),
    ],
    out_specs=pl.BlockSpec((FIN_BN, HEADS * OUT_CH), lambda i: (i, 0)),
)


def kernel(x, edge_index, W, a_src, a_dst, bias):
    xf = x.reshape(N_NODES)
    src = edge_index[0]
    dst = edge_index[1]
    Wr = W.reshape(HEADS, OUT_CH)
    s = jnp.sum(Wr * a_src, axis=1)
    d = jnp.sum(Wr * a_dst, axis=1)
    sv = jnp.broadcast_to(s[:, None], (HEADS, LANES))
    dv = jnp.broadcast_to(d[:, None], (HEADS, LANES))
    part = _sc_edge_pass(xf, src, dst, sv, dv)
    S = jnp.repeat(jnp.eye(HEADS, dtype=jnp.float32), OUT_CH, axis=1) * W
    out = _finalize(part, S, bias.reshape(1, HEADS * OUT_CH))
    return out


# speedup vs baseline: 407.5453x; 1.0003x over previous
import functools

import jax
import jax.numpy as jnp
from jax import lax
from jax.experimental import pallas as pl
from jax.experimental.pallas import tpu as pltpu
from jax.experimental.pallas import tpu_sc as plsc

N_NODES = 100000
N_EDGES = 1600000
HEADS = 8
OUT_CH = 8

NC = 2
NS = 16
NW = NC * NS
LANES = 16

BLK = 1280
CHUNK = 128
NCHUNK = BLK // CHUNK
NBLOCKS = N_EDGES // BLK
BLK_PER, BLK_REM = divmod(NBLOCKS, NW)
ROWS_PER_TILE = N_NODES // NS
NZCOPY = ROWS_PER_TILE // BLK
ZTAIL = ROWS_PER_TILE - NZCOPY * BLK


def _sc_body(x_hbm, src_hbm, dst_hbm, sv_hbm, dv_hbm, part_hbm,
             srcbuf, dstbuf, xsbuf, xdbuf, contrib, svv, dvv,
             acc, sem_g, sem_s):
    c = lax.axis_index("c")
    s = lax.axis_index("s")
    wid = s * NC + c

    pltpu.sync_copy(sv_hbm, svv)
    pltpu.sync_copy(dv_hbm, dvv)

    def _zero_rows(i, _):
        contrib[i, :] = jnp.zeros((LANES,), jnp.float32)
        return 0
    lax.fori_loop(0, BLK, _zero_rows, 0)

    def _zero_acc(j, _):
        pltpu.sync_copy(contrib,
                        acc.at[pl.ds(s * ROWS_PER_TILE + j * BLK, BLK)])
        return 0
    lax.fori_loop(0, NZCOPY, _zero_acc, 0)
    pltpu.sync_copy(contrib.at[pl.ds(0, ZTAIL)],
                    acc.at[pl.ds(s * ROWS_PER_TILE + NZCOPY * BLK, ZTAIL)])
    plsc.subcore_barrier()

    svh = [svv[h] for h in range(HEADS)]
    dvh = [dvv[h] for h in range(HEADS)]
    iota = lax.iota(jnp.int32, LANES)
    cols_d = [jnp.full((LANES,), h, jnp.int32) for h in range(HEADS)]
    cols_n = [jnp.full((LANES,), HEADS + h, jnp.int32) for h in range(HEADS)]

    nblk = BLK_PER + jnp.where(wid < BLK_REM, 1, 0)
    start = wid * BLK_PER + jnp.minimum(wid, BLK_REM)

    def _block(b, _):
        ebase = (start + b) * BLK
        cps = [pltpu.async_copy(src_hbm.at[pl.ds(ebase, BLK)], srcbuf, sem_g)]
        for k in range(NCHUNK):
            cps.append(pltpu.async_copy(
                dst_hbm.at[pl.ds(ebase + k * CHUNK, CHUNK)],
                dstbuf.at[k], sem_g))
        for cp in cps:
            cp.wait()
        cps = []
        for k in range(NCHUNK):
            cps.append(pltpu.async_copy(
                x_hbm.at[srcbuf.at[pl.ds(k * CHUNK, CHUNK)]],
                xsbuf.at[pl.ds(k * CHUNK, CHUNK)], sem_g))
            cps.append(pltpu.async_copy(
                x_hbm.at[dstbuf.at[k]],
                xdbuf.at[pl.ds(k * CHUNK, CHUNK)], sem_g))
        for cp in cps:
            cp.wait()

        def _group(g, _):
            xs = xsbuf[pl.ds(g * LANES, LANES)]
            xd = xdbuf[pl.ds(g * LANES, LANES)]
            row_idx = iota + g * LANES
            for h in range(HEADS):
                e = xs * svh[h] + xd * dvh[h]
                e = jnp.maximum(e, e * jnp.float32(0.2))
                ex = jnp.exp(e)
                plsc.store_scatter(contrib, [row_idx, cols_d[h]], ex)
                plsc.store_scatter(contrib, [row_idx, cols_n[h]], ex * xs)
            return 0

        lax.fori_loop(0, BLK // LANES, _group, 0)

        for k in range(NCHUNK):
            pltpu.async_copy(contrib.at[pl.ds(k * CHUNK, CHUNK)],
                             acc.at[dstbuf.at[k]], sem_s, add=True)
        pltpu.make_async_copy(contrib, acc.at[pl.ds(0, BLK)], sem_s).wait()
        return 0

    lax.fori_loop(0, nblk, _block, 0)
    plsc.subcore_barrier()

    pltpu.sync_copy(acc.at[pl.ds(s * ROWS_PER_TILE, ROWS_PER_TILE)],
                    part_hbm.at[c, pl.ds(s * ROWS_PER_TILE, ROWS_PER_TILE)])


@functools.partial(
    pl.kernel,
    mesh=plsc.VectorSubcoreMesh(core_axis_name="c", subcore_axis_name="s"),
    compiler_params=pltpu.CompilerParams(use_tc_tiling_on_sc=False,
                                         needs_layout_passes=False,
                                         skip_device_barrier=True),
    out_type=jax.ShapeDtypeStruct((NC, N_NODES, 2 * HEADS), jnp.float32),
    scratch_types=[
        pltpu.VMEM((BLK,), jnp.int32),
        pltpu.VMEM((NCHUNK, CHUNK), jnp.int32),
        pltpu.VMEM((BLK,), jnp.float32),
        pltpu.VMEM((BLK,), jnp.float32),
        pltpu.VMEM((BLK, 2 * HEADS), jnp.float32),
        pltpu.VMEM((HEADS, LANES), jnp.float32),
        pltpu.VMEM((HEADS, LANES), jnp.float32),
        pltpu.VMEM_SHARED((N_NODES, 2 * HEADS), jnp.float32),
        pltpu.SemaphoreType.DMA,
        pltpu.SemaphoreType.DMA,
    ],
)
def _sc_edge_pass(x_hbm, src_hbm, dst_hbm, sv_hbm, dv_hbm, part_hbm,
                  srcbuf, dstbuf, xsbuf, xdbuf, contrib, svv, dvv,
                  acc, sem_g, sem_s):
    _sc_body(x_hbm, src_hbm, dst_hbm, sv_hbm, dv_hbm, part_hbm,
             srcbuf, dstbuf, xsbuf, xdbuf, contrib, svv, dvv,
             acc, sem_g, sem_s)


FIN_BN = 1000


def _finalize_body(p_ref, s_ref, b_ref, o_ref):
    p = p_ref[0] + p_ref[1]
    den = p[:, :HEADS]
    num = p[:, HEADS:]
    g = num / (den + jnp.float32(1e-16))
    o = jnp.dot(g, s_ref[...], preferred_element_type=jnp.float32)
    o_ref[...] = jnp.maximum(o + b_ref[...], jnp.float32(0.0))


_finalize = pl.pallas_call(
    _finalize_body,
    out_shape=jax.ShapeDtypeStruct((N_NODES, HEADS * OUT_CH), jnp.float32),
    grid=(N_NODES // FIN_BN,),
    in_specs=[
        pl.BlockSpec((NC, FIN_BN, 2 * HEADS), lambda i: (0, i, 0)),
        pl.BlockSpec((HEADS, HEADS * OUT_CH), lambda i: (0, 0)),
        pl.BlockSpec((1, HEADS * OUT_CH), lambda i: (0, 0)),
    ],
    out_specs=pl.BlockSpec((FIN_BN, HEADS * OUT_CH), lambda i: (i, 0)),
)


def kernel(x, edge_index, W, a_src, a_dst, bias):
    xf = x.reshape(N_NODES)
    src = edge_index[0]
    dst = edge_index[1]
    Wr = W.reshape(HEADS, OUT_CH)
    s = jnp.sum(Wr * a_src, axis=1)
    d = jnp.sum(Wr * a_dst, axis=1)
    sv = jnp.broadcast_to(s[:, None], (HEADS, LANES))
    dv = jnp.broadcast_to(d[:, None], (HEADS, LANES))
    part = _sc_edge_pass(xf, src, dst, sv, dv)
    S = jnp.repeat(jnp.eye(HEADS, dtype=jnp.float32), OUT_CH, axis=1) * W
    out = _finalize(part, S, bias.reshape(1, HEADS * OUT_CH))
    return out
